# baseline (speedup 1.0000x reference)
import jax
import jax.numpy as jnp
from jax import lax
from jax.experimental import pallas as pl
from jax.experimental.pallas import tpu as pltpu

B = 128
IN = 4096
HID = 4096
OUT = 1024
EP = 64
KP = 1024
NC = 1024
NP = IN // KP

L1_N = (HID // NC) * NP
L2_I = L1_N
L3_0 = L2_I + 1
L4_0 = L3_0 + 16
L5_0 = L4_0 + 16
STEPS = L5_0 + NP

_F32 = jnp.float32
_BF16 = jnp.bfloat16
_DN = (((1,), (0,)), ((), ()))


def _mdot(a, b):
    return lax.dot_general(a, b.astype(_BF16), _DN,
                           preferred_element_type=_F32)


def _body(x_ref, wenc1_ref, benc1_ref, wenc2_ref, benc2_ref,
          w1m_ref, w1t_ref, b1_ref, w2_ref, b2_ref, w3_ref, b3_ref,
          out_ref, xb, henc, enc, h1, h2, acc):
    i = pl.program_id(0)

    @pl.when(i == 0)
    def _cast_x():
        xb[...] = x_ref[...].astype(_BF16)

    @pl.when(i < L1_N)
    def _l1():
        p = i % NP
        j = i // NP
        part = _mdot(xb[:, pl.ds(p * KP, KP)], wenc1_ref[...])

        @pl.when(p == 0)
        def _():
            acc[...] = part + benc1_ref[...]

        @pl.when(jnp.logical_and(p > 0, p < NP - 1))
        def _():
            acc[...] += part

        @pl.when(p == NP - 1)
        def _():
            henc[:, pl.ds(j * NC, NC)] = jnp.maximum(acc[...] + part,
                                                     0.0).astype(_BF16)

    @pl.when(i == L2_I)
    def _l2():
        e = _mdot(henc[...], wenc2_ref[...])
        enc[...] = (e + benc2_ref[...]).astype(_BF16)

    @pl.when(jnp.logical_and(i >= L3_0, i < L4_0))
    def _l3():
        s = i - L3_0
        p = s % NP
        j = s // NP
        part = _mdot(xb[:, pl.ds(p * KP, KP)], w1m_ref[...])

        @pl.when(p == 0)
        def _():
            acc[...] = part + b1_ref[...] + _mdot(enc[...], w1t_ref[...])

        @pl.when(jnp.logical_and(p > 0, p < NP - 1))
        def _():
            acc[...] += part

        @pl.when(p == NP - 1)
        def _():
            h1[:, pl.ds(j * NC, NC)] = jnp.maximum(acc[...] + part,
                                                   0.0).astype(_BF16)

    @pl.when(jnp.logical_and(i >= L4_0, i < L5_0))
    def _l4():
        s = i - L4_0
        p = s % NP
        j = s // NP
        part = _mdot(h1[:, pl.ds(p * KP, KP)], w2_ref[...])

        @pl.when(p == 0)
        def _():
            acc[...] = part + b2_ref[...]

        @pl.when(jnp.logical_and(p > 0, p < NP - 1))
        def _():
            acc[...] += part

        @pl.when(p == NP - 1)
        def _():
            h2[:, pl.ds(j * NC, NC)] = jnp.maximum(acc[...] + part,
                                                   0.0).astype(_BF16)

    @pl.when(i >= L5_0)
    def _l5():
        p = i - L5_0
        part = _mdot(h2[:, pl.ds(p * KP, KP)], w3_ref[...])

        @pl.when(p == 0)
        def _():
            acc[...] = part + b3_ref[...]

        @pl.when(jnp.logical_and(p > 0, p < NP - 1))
        def _():
            acc[...] += part

        @pl.when(p == NP - 1)
        def _():
            out_ref[...] = acc[...] + part


def _pj1(i):
    c = jnp.clip(i, 0, L1_N - 1)
    return c % NP, c // NP


def _pj3(i):
    c = jnp.clip(i - L3_0, 0, 15)
    return c % NP, c // NP


def _pj4(i):
    c = jnp.clip(i - L4_0, 0, 15)
    return c % NP, c // NP


def _p5(i):
    return jnp.clip(i - L5_0, 0, NP - 1)


def kernel(x, W_enc1, b_enc1, W_enc2, b_enc2, W1, b1, W2, b2, W3, b3):
    benc1 = b_enc1.reshape(1, HID)
    benc2 = b_enc2.reshape(1, EP)
    b1r = b1.reshape(1, HID)
    b2r = b2.reshape(1, HID)
    b3r = b3.reshape(1, OUT)

    in_specs = [
        pl.BlockSpec((B, IN), lambda i: (0, 0)),
        pl.BlockSpec((KP, NC), lambda i: _pj1(i)),
        pl.BlockSpec((1, NC), lambda i: (0, _pj1(i)[1])),
        pl.BlockSpec((HID, EP), lambda i: (0, 0)),
        pl.BlockSpec((1, EP), lambda i: (0, 0)),
        pl.BlockSpec((KP, NC), lambda i: _pj3(i)),
        pl.BlockSpec((EP, NC), lambda i: (IN // EP, _pj3(i)[1])),
        pl.BlockSpec((1, NC), lambda i: (0, _pj3(i)[1])),
        pl.BlockSpec((KP, NC), lambda i: _pj4(i)),
        pl.BlockSpec((1, NC), lambda i: (0, _pj4(i)[1])),
        pl.BlockSpec((KP, OUT), lambda i: (_p5(i), 0)),
        pl.BlockSpec((1, OUT), lambda i: (0, 0)),
    ]
    out_spec = pl.BlockSpec((B, OUT), lambda i: (0, 0))

    return pl.pallas_call(
        _body,
        grid=(STEPS,),
        in_specs=in_specs,
        out_specs=out_spec,
        out_shape=jax.ShapeDtypeStruct((B, OUT), _F32),
        scratch_shapes=[
            pltpu.VMEM((B, IN), _BF16),
            pltpu.VMEM((B, HID), _BF16),
            pltpu.VMEM((B, EP), _BF16),
            pltpu.VMEM((B, HID), _BF16),
            pltpu.VMEM((B, HID), _BF16),
            pltpu.VMEM((B, NC), _F32),
        ],
        compiler_params=pltpu.CompilerParams(
            dimension_semantics=("arbitrary",),
        ),
    )(x, W_enc1, benc1, W_enc2, benc2, W1, W1, b1r, W2, b2r, W3, b3r)


# speedup vs baseline: 1.1326x; 1.1326x over previous
import jax
import jax.numpy as jnp
from jax import lax
from jax.experimental import pallas as pl
from jax.experimental.pallas import tpu as pltpu

B = 128
IN = 4096
HID = 4096
OUT = 1024
EP = 64
KP = 1024
NC = 1024
NP = IN // KP

PA_N = (HID // NC) * NP
PB_I = PA_N
PC_0 = PB_I + 1
STEPS = PC_0 + PA_N

_F32 = jnp.float32
_BF16 = jnp.bfloat16
_DN = (((1,), (0,)), ((), ()))


def _mdot(a, b):
    return lax.dot_general(a, b.astype(_BF16), _DN,
                           preferred_element_type=_F32)


def _body(x_ref, wenc1_ref, benc1_ref, wenc2_ref, benc2_ref,
          w1m_ref, w1t_ref, b1_ref, w2_ref, b2_ref, w3_ref, b3_ref,
          out_ref, xb, henc, h1pre, h1, h2, acce, accc):
    i = pl.program_id(0)

    @pl.when(i == 0)
    def _cast_x():
        xb[...] = x_ref[...].astype(_BF16)

    @pl.when(i < PA_N)
    def _pa():
        p = i % NP
        j = i // NP
        xs = xb[:, pl.ds(p * KP, KP)]
        pe = _mdot(xs, wenc1_ref[...])
        p1 = _mdot(xs, w1m_ref[...])

        @pl.when(p == 0)
        def _():
            acce[...] = pe + benc1_ref[...]
            h1pre[:, pl.ds(j * NC, NC)] = p1 + b1_ref[...]

        @pl.when(jnp.logical_and(p > 0, p < NP - 1))
        def _():
            acce[...] += pe
            h1pre[:, pl.ds(j * NC, NC)] += p1

        @pl.when(p == NP - 1)
        def _():
            henc[:, pl.ds(j * NC, NC)] = jnp.maximum(acce[...] + pe,
                                                     0.0).astype(_BF16)
            h1pre[:, pl.ds(j * NC, NC)] += p1

    @pl.when(i == PB_I)
    def _pb():
        e = _mdot(henc[...], wenc2_ref[...])
        enc = (e + benc2_ref[...]).astype(_BF16)
        h1[...] = jnp.maximum(h1pre[...] + _mdot(enc, w1t_ref[...]),
                              0.0).astype(_BF16)

    @pl.when(i >= PC_0)
    def _pc():
        s = i - PC_0
        p = s % NP
        j = s // NP
        p2 = _mdot(h1[:, pl.ds(p * KP, KP)], w2_ref[...])

        @pl.when(p == 0)
        def _():
            accc[...] = p2 + b2_ref[...]

        @pl.when(jnp.logical_and(p > 0, p < NP - 1))
        def _():
            accc[...] += p2

        @pl.when(p == NP - 1)
        def _():
            h2col = jnp.maximum(accc[...] + p2, 0.0).astype(_BF16)
            h2[...] = h2col
            po = _mdot(h2col, w3_ref[...])

            @pl.when(j == 0)
            def _():
                out_ref[...] = po + b3_ref[...]

            @pl.when(j > 0)
            def _():
                out_ref[...] += po


def _pja(i):
    c = jnp.clip(i, 0, PA_N - 1)
    return c % NP, c // NP


def _pjc(i):
    c = jnp.clip(i - PC_0, 0, PA_N - 1)
    return c % NP, c // NP


def _jc(i):
    return jnp.clip(i - PC_0, 0, PA_N - 1) // NP


def kernel(x, W_enc1, b_enc1, W_enc2, b_enc2, W1, b1, W2, b2, W3, b3):
    benc1 = b_enc1.reshape(1, HID)
    benc2 = b_enc2.reshape(1, EP)
    b1r = b1.reshape(1, HID)
    b2r = b2.reshape(1, HID)
    b3r = b3.reshape(1, OUT)

    in_specs = [
        pl.BlockSpec((B, IN), lambda i: (0, 0)),
        pl.BlockSpec((KP, NC), lambda i: _pja(i)),
        pl.BlockSpec((1, NC), lambda i: (0, _pja(i)[1])),
        pl.BlockSpec((HID, EP), lambda i: (0, 0)),
        pl.BlockSpec((1, EP), lambda i: (0, 0)),
        pl.BlockSpec((KP, NC), lambda i: _pja(i)),
        pl.BlockSpec((EP, HID), lambda i: (IN // EP, 0)),
        pl.BlockSpec((1, NC), lambda i: (0, _pja(i)[1])),
        pl.BlockSpec((KP, NC), lambda i: _pjc(i)),
        pl.BlockSpec((1, NC), lambda i: (0, _pjc(i)[1])),
        pl.BlockSpec((KP, OUT), lambda i: (_jc(i), 0)),
        pl.BlockSpec((1, OUT), lambda i: (0, 0)),
    ]
    out_spec = pl.BlockSpec((B, OUT), lambda i: (0, 0))

    return pl.pallas_call(
        _body,
        grid=(STEPS,),
        in_specs=in_specs,
        out_specs=out_spec,
        out_shape=jax.ShapeDtypeStruct((B, OUT), _F32),
        scratch_shapes=[
            pltpu.VMEM((B, IN), _BF16),
            pltpu.VMEM((B, HID), _BF16),
            pltpu.VMEM((B, HID), _F32),
            pltpu.VMEM((B, HID), _BF16),
            pltpu.VMEM((B, NC), _BF16),
            pltpu.VMEM((B, NC), _F32),
            pltpu.VMEM((B, NC), _F32),
        ],
        compiler_params=pltpu.CompilerParams(
            dimension_semantics=("arbitrary",),
        ),
    )(x, W_enc1, benc1, W_enc2, benc2,
      W1, W1, b1r, W2, b2r, W3, b3r)
